# Initial kernel scaffold; baseline (speedup 1.0000x reference)
import jax
import jax.numpy as jnp
from jax.experimental import pallas as pl


def kernel(x, edge_index, W, a):
    raise NotImplementedError("write your pallas kernel here")


# speedup vs baseline: 4.7198x; 4.7198x over previous
import functools

import jax
import jax.numpy as jnp
from jax import lax
from jax.experimental import pallas as pl
from jax.experimental.pallas import tpu as pltpu
from jax.experimental.pallas import tpu_sc as plsc

N = 10000
E = 320000
D = 128
H = 4
DH = D // H
ALPHA = 0.2
EPS = 1e-15

NC = 2
NS = 16
NW = NC * NS
EPT = E // NW
ROW = 80
NCHUNK = EPT // ROW
NPAD = 10240
RPS = NPAD // NS
G16 = ROW // 16


def _prep_body(x_ref, wall_ref, ab_ref, ht_ref, st_ref):
  x = x_ref[...]
  nrm = jnp.maximum(jnp.sqrt(jnp.sum(x * x, axis=1, keepdims=True)), EPS)
  r = jnp.clip(nrm, -1.0 + 1e-5, 1.0 - 1e-5)
  at = 0.5 * (jnp.log1p(r) - jnp.log1p(-r))
  xt = x / nrm * at
  h = jnp.dot(xt, wall_ref[...], preferred_element_type=jnp.float32)
  ht_ref[...] = h
  st_ref[...] = jnp.dot(h, ab_ref[...], preferred_element_type=jnp.float32)


def _post_body(hp_ref, rs_ref, rep_ref, out_ref, rs8_ref):
  acc = (hp_ref[0] + hp_ref[1])[:N]
  rsum4 = (rs_ref[0:4] + rs_ref[4:8])[:, :N]
  rs8_ref[...] = jnp.concatenate(
      [rsum4, jnp.zeros((8 - H, N), jnp.float32)], axis=0)
  den = lax.dot_general(
      rsum4 + 1e-16, rep_ref[...], (((0,), (0,)), ((), ())),
      preferred_element_type=jnp.float32)
  sup = acc / den
  sup = jnp.where(sup > 0, sup, jnp.exp(jnp.minimum(sup, 0.0)) - 1.0)
  snrm = jnp.maximum(jnp.sqrt(jnp.sum(sup * sup, axis=1, keepdims=True)), EPS)
  ex = jnp.tanh(snrm) * sup / snrm
  enrm = jnp.maximum(jnp.sqrt(jnp.sum(ex * ex, axis=1, keepdims=True)), EPS)
  maxn = 1.0 - 4e-3
  out_ref[...] = jnp.where(enrm > maxn, ex / enrm * maxn, ex)


def _edge_body(ht_hbm, st_hbm, src_hbm, dst_hbm,
               ee_out, hp_out, rs_out,
               svals_s, svals_d, src_v, dst_v, rows, ebuf,
               ridx0, ridx1, ridx2, ridx3, hp_sh, rs_sh, sem):
  c = lax.axis_index("c")
  s = lax.axis_index("s")
  wid = c * NS + s
  iota = lax.iota(jnp.int32, 16)
  zeros16 = jnp.zeros((16,), jnp.float32)
  ridx = [ridx0, ridx1, ridx2, ridx3]

  for r in range(ROW):
    for k in range(D // 16):
      rows[r, pl.ds(k * 16, 16)] = zeros16
  for k in range(H * ROW // 16):
    ebuf[pl.ds(k * 16, 16)] = zeros16
  for k in range(RPS // ROW):
    pltpu.sync_copy(rows, hp_sh.at[pl.ds(s * RPS + k * ROW, ROW)])
  for k in range(RPS // ROW):
    pltpu.sync_copy(ebuf, rs_sh.at[pl.ds(s * H * RPS + k * H * ROW, H * ROW)])

  plsc.subcore_barrier()

  def chunk_body(ch, _):
    ebase = wid * EPT + ch * ROW
    pltpu.sync_copy(src_hbm.at[pl.ds(ebase, ROW)], src_v)
    pltpu.sync_copy(dst_hbm.at[pl.ds(ebase, ROW)], dst_v)
    cp0 = pltpu.async_copy(ht_hbm.at[dst_v], rows, sem)
    cp1 = pltpu.async_copy(st_hbm.at[src_v], svals_s, sem)
    cp2 = pltpu.async_copy(st_hbm.at[dst_v], svals_d, sem)
    cp0.wait()
    cp1.wait()
    cp2.wait()

    def g_body(g, _):
      off = g * 16
      e16 = iota + off
      src16 = src_v[pl.ds(off, 16)]
      for hh in range(H):
        ssrc = plsc.load_gather(svals_s, [e16, jnp.full((16,), hh, jnp.int32)])
        sdst = plsc.load_gather(svals_d,
                                [e16, jnp.full((16,), H + hh, jnp.int32)])
        lg = ssrc + sdst
        ee = jnp.exp(-jnp.maximum(lg, ALPHA * lg))
        ebuf[pl.ds(hh * ROW + off, 16)] = ee
        ridx[hh][pl.ds(off, 16)] = src16 + hh * NPAD
        for j in range(DH):
          cv = jnp.full((16,), hh * DH + j, jnp.int32)
          v = plsc.load_gather(rows, [e16, cv])
          plsc.store_scatter(rows, [e16, cv], v * ee)
      return 0

    lax.fori_loop(0, G16, g_body, 0)

    pltpu.sync_copy(rows, hp_sh.at[src_v], add=True)
    for hh in range(H):
      pltpu.sync_copy(ebuf.at[pl.ds(hh * ROW, ROW)],
                      rs_sh.at[ridx[hh]], add=True)
      pltpu.sync_copy(ebuf.at[pl.ds(hh * ROW, ROW)],
                      ee_out.at[pl.ds(hh * E + ebase, ROW)])
    return 0

  lax.fori_loop(0, NCHUNK, chunk_body, 0)

  plsc.subcore_barrier()
  pltpu.sync_copy(hp_sh.at[pl.ds(s * RPS, RPS)],
                  hp_out.at[c, pl.ds(s * RPS, RPS)])
  pltpu.sync_copy(rs_sh.at[pl.ds(s * H * RPS, H * RPS)],
                  rs_out.at[c, 0, pl.ds(s * H * RPS, H * RPS)])


@functools.cache
def _edge_kernel():
  return functools.partial(
      pl.kernel,
      out_type=(jax.ShapeDtypeStruct((H * E,), jnp.float32),
                jax.ShapeDtypeStruct((NC, NPAD, D), jnp.float32),
                jax.ShapeDtypeStruct((NC, 1, H * NPAD), jnp.float32)),
      mesh=plsc.VectorSubcoreMesh(core_axis_name="c", subcore_axis_name="s",
                                  num_cores=NC, num_subcores=NS),
      compiler_params=pltpu.CompilerParams(use_tc_tiling_on_sc=False,
                                           needs_layout_passes=False),
      scratch_types=[
          pltpu.VMEM((ROW, 2 * H), jnp.float32),
          pltpu.VMEM((ROW, 2 * H), jnp.float32),
          pltpu.VMEM((ROW,), jnp.int32),
          pltpu.VMEM((ROW,), jnp.int32),
          pltpu.VMEM((ROW, D), jnp.float32),
          pltpu.VMEM((H * ROW,), jnp.float32),
          pltpu.VMEM((ROW,), jnp.int32),
          pltpu.VMEM((ROW,), jnp.int32),
          pltpu.VMEM((ROW,), jnp.int32),
          pltpu.VMEM((ROW,), jnp.int32),
          pltpu.VMEM_SHARED((NPAD, D), jnp.float32),
          pltpu.VMEM_SHARED((H * NPAD,), jnp.float32),
          pltpu.SemaphoreType.DMA,
      ],
  )(_edge_body)


def kernel(x, edge_index, W, a):
  wall = jnp.transpose(W, (1, 0, 2)).reshape(D, D)
  ab = jnp.zeros((D, 2 * H), jnp.float32)
  for hh in range(H):
    ab = ab.at[hh * DH:(hh + 1) * DH, hh].set(a[hh, :DH])
    ab = ab.at[hh * DH:(hh + 1) * DH, H + hh].set(a[hh, DH:])
  src1d = edge_index[0]
  dst1d = edge_index[1]

  ht, st = pl.pallas_call(
      _prep_body,
      out_shape=(jax.ShapeDtypeStruct((N, D), jnp.float32),
                 jax.ShapeDtypeStruct((N, 2 * H), jnp.float32)),
  )(x, wall, ab)

  ee_flat, hp, rs = _edge_kernel()(ht, st, src1d, dst1d)

  rep = jnp.zeros((H, D), jnp.float32)
  for hh in range(H):
    rep = rep.at[hh, hh * DH:(hh + 1) * DH].set(1.0)

  out, rs8 = pl.pallas_call(
      _post_body,
      out_shape=(jax.ShapeDtypeStruct((N, D), jnp.float32),
                 jax.ShapeDtypeStruct((8, N), jnp.float32)),
  )(hp, rs.reshape(NC * H, NPAD), rep)

  return out, ee_flat.reshape(H, E), rs8[:H]
